# baseline (speedup 1.0000x reference)
import functools

import jax
import jax.numpy as jnp
from jax import lax
from jax.experimental import pallas as pl
from jax.experimental.pallas import tpu as pltpu
from jax.experimental.pallas import tpu_sc as plsc

_P = 0.15
_B, _F = 16384, 100
_N = _B * _F
_NC, _NS = 2, 16
_NW = _NC * _NS
_RPW = _B // _NW
_CH_R = 64
_NCHUNK = _RPW // _CH_R
_CHUNK = _CH_R * _F
_G = 128
_NG = _CHUNK // _G
_VPR = 7
_IL = 8


@functools.partial(
    pl.kernel,
    out_type=jax.ShapeDtypeStruct((_B, _F), jnp.float32),
    mesh=plsc.VectorSubcoreMesh(core_axis_name="c", subcore_axis_name="s"),
    compiler_params=pltpu.CompilerParams(
        needs_layout_passes=False, use_tc_tiling_on_sc=True),
    scratch_types=[
        pltpu.VMEM((_CH_R, _F), jnp.float32),
        pltpu.VMEM((_CH_R, _F), jnp.float32),
        pltpu.VMEM((_CH_R, _F), jnp.float32),
        pltpu.VMEM((_CH_R, _F), jnp.float32),
        pltpu.VMEM((_CH_R + 8, _F), jnp.float32),
        pltpu.VMEM((_CH_R + 8, _F), jnp.float32),
        pltpu.VMEM((_NG + 1, _G), jnp.int32),
        pltpu.VMEM((_CHUNK + _G,), jnp.int32),
        pltpu.VMEM((_NG, _G), jnp.float32),
        pltpu.SemaphoreType.DMA,
        pltpu.SemaphoreType.DMA,
        pltpu.SemaphoreType.DMA,
    ],
)
def _swap_noise(xf_hbm, x_hbm, rm_hbm, rr_hbm, out_hbm,
                rm_v0, rm_v1, rr_v0, rr_v1, ov0, ov1,
                gidx_v, pos_v, gath_v, sem, sem_in, sem_out):
    wid = lax.axis_index("s") * _NC + lax.axis_index("c")
    rbase = wid * _RPW
    iota = lax.iota(jnp.int32, 16)
    thresh = jnp.float32(1.0 - _P)
    bf = jnp.float32(_B)
    zeros_i = jnp.zeros((16,), jnp.int32)
    valid6 = iota >= (6 * 16 - (_F - 16))
    bufs = [(rm_v0, rr_v0, ov0), (rm_v1, rr_v1, ov1)]

    def fire_loads(cc, b):
        rmc, rrc, ovc = b
        r0 = pl.multiple_of(rbase + cc * _CH_R, _CH_R)
        pltpu.async_copy(
            x_hbm.at[pl.ds(r0, _CH_R)], ovc.at[pl.ds(0, _CH_R)], sem_in)
        pltpu.async_copy(rm_hbm.at[pl.ds(r0, _CH_R)], rmc, sem_in)
        pltpu.async_copy(rr_hbm.at[pl.ds(r0, _CH_R)], rrc, sem_in)

    def wait_loads(b):
        rmc, rrc, ovc = b
        pltpu.make_async_copy(
            x_hbm.at[pl.ds(0, _CH_R)], ovc.at[pl.ds(0, _CH_R)], sem_in).wait()
        pltpu.make_async_copy(rm_hbm.at[pl.ds(0, _CH_R)], rmc, sem_in).wait()
        pltpu.make_async_copy(rr_hbm.at[pl.ds(0, _CH_R)], rrc, sem_in).wait()

    def wait_out(b):
        ovc = b[2]
        pltpu.make_async_copy(
            ovc.at[pl.ds(0, _CH_R)],
            out_hbm.at[pl.ds(pl.multiple_of(rbase, _CH_R), _CH_R)],
            sem_out).wait()

    fire_loads(0, bufs[0])

    for c in range(_NCHUNK):
        rmc, rrc, ovc = bufs[c & 1]
        row0 = pl.multiple_of(rbase + c * _CH_R, _CH_R)
        wait_loads(bufs[c & 1])
        if c > 0:
            wait_out(bufs[(c + 1) & 1])
        if c + 1 < _NCHUNK:
            fire_loads(c + 1, bufs[(c + 1) & 1])

        def grp_body(rq, acc, rmc=rmc, rrc=rrc, row0=row0):
            rows = [rq * _IL + d for d in range(_IL)]
            for u in range(_VPR):
                c0 = u * 16 if u < _VPR - 1 else _F - 16
                cols = c0 + iota
                rms = [rmc[r, pl.ds(c0, 16)] for r in rows]
                rrs = [rrc[r, pl.ds(c0, 16)] for r in rows]
                ms = [rm > thresh for rm in rms]
                if u == _VPR - 1:
                    ms = [jnp.logical_and(m, valid6) for m in ms]
                mis = [jnp.where(m, 1, 0) for m in ms]
                pcs = [plsc.all_reduce_population_count(m) for m in ms]
                pfxs = [plsc.cumsum(mi) - mi for mi in mis]
                ss = [(rr * bf).astype(jnp.int32) for rr in rrs]
                idxs = [((row0 + r + s) & (_B - 1)) * _F + cols
                        for r, s in zip(rows, ss)]
                dsts = []
                for d in range(_IL):
                    dsts.append(acc + pfxs[d])
                    acc = acc + pcs[d]
                for d in range(_IL):
                    plsc.store_scatter(
                        gidx_v, [dsts[d] >> 7, dsts[d] & 127], idxs[d],
                        mask=ms[d])
                    plsc.store_scatter(
                        pos_v, [dsts[d]], rows[d] * 128 + cols, mask=ms[d])
            return acc

        acc = lax.fori_loop(0, _CH_R // _IL, grp_body, zeros_i)
        cnt = acc[0]

        gbase = (cnt >> 7) << 7
        for u in range(8):
            lp = (gbase + u * 16) + iota
            tm = lp >= cnt
            plsc.store_scatter(gidx_v, [lp >> 7, lp & 127], zeros_i, mask=tm)
            plsc.store_scatter(pos_v, [lp], (_CH_R << 7) + iota, mask=tm)

        ng = (cnt + 127) >> 7

        def fire(g, carry2):
            pltpu.async_copy(xf_hbm.at[gidx_v.at[g]], gath_v.at[g], sem)
            return carry2

        lax.fori_loop(0, ng, fire, 0)

        def drain(g, carry2):
            pltpu.make_async_copy(
                xf_hbm.at[gidx_v.at[g]], gath_v.at[g], sem).wait()
            return carry2

        lax.fori_loop(0, ng, drain, 0)

        def scatter_back(g, carry2, ovc=ovc):
            for u in range(8):
                val = gath_v[g, pl.ds(u * 16, 16)]
                pvec = pos_v[pl.ds(g * _G + u * 16, 16)]
                plsc.store_scatter(ovc, [pvec >> 7, pvec & 127], val)
            return carry2

        lax.fori_loop(0, ng, scatter_back, 0)
        pltpu.async_copy(
            ovc.at[pl.ds(0, _CH_R)], out_hbm.at[pl.ds(row0, _CH_R)], sem_out)

    wait_out(bufs[(_NCHUNK - 1) & 1])


def kernel(x, rand_mask, rand_rows):
    return _swap_noise(x.reshape(-1), x, rand_mask, rand_rows)


# speedup vs baseline: 1.2949x; 1.2949x over previous
import functools

import jax
import jax.numpy as jnp
from jax import lax
from jax.experimental import pallas as pl
from jax.experimental.pallas import tpu as pltpu
from jax.experimental.pallas import tpu_sc as plsc

_P = 0.15
_B, _F = 16384, 100
_N = _B * _F
_NC, _NS = 2, 16
_NW = _NC * _NS
_RPW = _B // _NW
_CH_R = 128
_NCHUNK = _RPW // _CH_R
_CHUNK = _CH_R * _F
_G = 128
_NG = _CHUNK // _G
_VPR = 7


@functools.partial(
    pl.kernel,
    out_type=jax.ShapeDtypeStruct((_B, _F), jnp.float32),
    mesh=plsc.VectorSubcoreMesh(core_axis_name="c", subcore_axis_name="s"),
    compiler_params=pltpu.CompilerParams(
        needs_layout_passes=False, use_tc_tiling_on_sc=True),
    scratch_types=[
        pltpu.VMEM((_CH_R, _F), jnp.float32),
        pltpu.VMEM((_CH_R, _F), jnp.float32),
        pltpu.VMEM((_CH_R + 8, _F), jnp.float32),
        pltpu.VMEM((_NG + 1, _G), jnp.int32),
        pltpu.VMEM((_CHUNK + _G,), jnp.int32),
        pltpu.VMEM((_NG, _G), jnp.float32),
        pltpu.SemaphoreType.DMA,
        pltpu.SemaphoreType.DMA,
    ],
)
def _swap_noise(xf_hbm, x_hbm, rm_hbm, rr_hbm, out_hbm,
                rm_v, rr_v, out_v, gidx_v, pos_v, gath_v, sem, sem2):
    wid = lax.axis_index("s") * _NC + lax.axis_index("c")
    rbase = wid * _RPW
    iota = lax.iota(jnp.int32, 16)
    thresh = jnp.float32(1.0 - _P)
    bf = jnp.float32(_B)
    zeros_i = jnp.zeros((16,), jnp.int32)
    valid6 = iota >= (6 * 16 - (_F - 16))

    def chunk_body(c, carry):
        row0 = pl.multiple_of(rbase + c * _CH_R, _CH_R)
        idcp = pltpu.async_copy(
            x_hbm.at[pl.ds(row0, _CH_R)], out_v.at[pl.ds(0, _CH_R)], sem2)
        pltpu.sync_copy(rm_hbm.at[pl.ds(row0, _CH_R)], rm_v)
        pltpu.sync_copy(rr_hbm.at[pl.ds(row0, _CH_R)], rr_v)

        def quad_body(rq, acc):
            rows = [rq * 8 + d for d in range(8)]
            for u in range(_VPR):
                c0 = u * 16 if u < _VPR - 1 else _F - 16
                cols = c0 + iota
                rms = [rm_v[r, pl.ds(c0, 16)] for r in rows]
                rrs = [rr_v[r, pl.ds(c0, 16)] for r in rows]
                ms = [rm > thresh for rm in rms]
                if u == _VPR - 1:
                    ms = [jnp.logical_and(m, valid6) for m in ms]
                mis = [jnp.where(m, 1, 0) for m in ms]
                pcs = [plsc.all_reduce_population_count(m) for m in ms]
                pfxs = [plsc.cumsum(mi) - mi for mi in mis]
                ss = [(rr * bf).astype(jnp.int32) for rr in rrs]
                idxs = [((row0 + r + s) & (_B - 1)) * _F + cols
                        for r, s in zip(rows, ss)]
                dsts = []
                for d in range(8):
                    dsts.append(acc + pfxs[d])
                    acc = acc + pcs[d]
                for d in range(8):
                    plsc.store_scatter(
                        gidx_v, [dsts[d] >> 7, dsts[d] & 127], idxs[d],
                        mask=ms[d])
                    plsc.store_scatter(
                        pos_v, [dsts[d]], rows[d] * 128 + cols, mask=ms[d])
            return acc

        acc = lax.fori_loop(0, _CH_R // 8, quad_body, zeros_i)
        cnt = acc[0]

        gbase = (cnt >> 7) << 7
        for u in range(8):
            lp = (gbase + u * 16) + iota
            tm = lp >= cnt
            plsc.store_scatter(gidx_v, [lp >> 7, lp & 127], zeros_i, mask=tm)
            plsc.store_scatter(pos_v, [lp], (_CH_R << 7) + iota, mask=tm)

        ng = (cnt + 127) >> 7

        def fire(g, carry2):
            pltpu.async_copy(xf_hbm.at[gidx_v.at[g]], gath_v.at[g], sem)
            return carry2

        lax.fori_loop(0, ng, fire, 0)
        idcp.wait()

        def drain(g, carry2):
            pltpu.make_async_copy(
                xf_hbm.at[gidx_v.at[g]], gath_v.at[g], sem).wait()
            return carry2

        lax.fori_loop(0, ng, drain, 0)

        def scatter_back(g, carry2):
            for u in range(8):
                val = gath_v[g, pl.ds(u * 16, 16)]
                pvec = pos_v[pl.ds(g * _G + u * 16, 16)]
                plsc.store_scatter(out_v, [pvec >> 7, pvec & 127], val)
            return carry2

        lax.fori_loop(0, ng, scatter_back, 0)
        pltpu.sync_copy(out_v.at[pl.ds(0, _CH_R)], out_hbm.at[pl.ds(row0, _CH_R)])
        return carry

    lax.fori_loop(0, _NCHUNK, chunk_body, 0)


def kernel(x, rand_mask, rand_rows):
    return _swap_noise(x.reshape(-1), x, rand_mask, rand_rows)
